# baseline (speedup 1.0000x reference)
import functools
import jax
import jax.numpy as jnp
from jax import lax
from jax.experimental import pallas as pl
from jax.experimental.pallas import tpu as pltpu
from jax.experimental.pallas import tpu_sc as plsc

_B = 16384
_D = 64
_NC = 2
_NS = 16
_NW = _NC * _NS
_BPW = _B // _NW
_CHUNK = 128
_NCHUNK = _BPW // _CHUNK
_LANES = 16


def _sc_body(feat_hbm, lab_hbm, cent_hbm, out_hbm, idx_v, rows_v, feat_v,
             acc_v, gsem):
    wid = lax.axis_index("s") * _NC + lax.axis_index("c")
    base = wid * _BPW

    pltpu.sync_copy(lab_hbm.at[wid], idx_v)

    copies = []
    for j in range(_NCHUNK):
        copies.append(
            pltpu.async_copy(
                cent_hbm.at[idx_v.at[j]],
                rows_v.at[pl.ds(j * _CHUNK, _CHUNK)],
                gsem,
            )
        )
    pltpu.sync_copy(feat_hbm.at[pl.ds(base, _BPW)], feat_v)
    for c in copies:
        c.wait()

    def row_body(r, acc):
        for c in range(_D // _LANES):
            f = feat_v[r, pl.ds(c * _LANES, _LANES)]
            ce = rows_v[r, pl.ds(c * _LANES, _LANES)]
            d = f - ce
            acc = acc + d * d
        return acc

    acc = lax.fori_loop(0, _BPW, row_body, jnp.zeros((_LANES,), jnp.float32))
    acc_v[...] = acc * jnp.float32(1.0 / (_B * _D))
    pltpu.sync_copy(acc_v, out_hbm.at[wid])


@jax.jit
def _center_loss_sc(features, labels_r, centers):
    mesh = plsc.VectorSubcoreMesh(
        core_axis_name="c", subcore_axis_name="s",
        num_cores=_NC, num_subcores=_NS,
    )
    partials = pl.kernel(
        _sc_body,
        out_type=jax.ShapeDtypeStruct((_NW, _LANES), jnp.float32),
        mesh=mesh,
        scratch_types=[
            pltpu.VMEM((_NCHUNK, _CHUNK), jnp.int32),
            pltpu.VMEM((_BPW, _D), jnp.float32),
            pltpu.VMEM((_BPW, _D), jnp.float32),
            pltpu.VMEM((_LANES,), jnp.float32),
            pltpu.SemaphoreType.DMA,
        ],
        compiler_params=pltpu.CompilerParams(use_tc_tiling_on_sc=False),
    )(features, labels_r, centers)
    return jnp.sum(partials)


def kernel(features, labels, centers):
    labels_r = labels.astype(jnp.int32).reshape(_NW, _NCHUNK, _CHUNK)
    return _center_loss_sc(features, labels_r, centers)
